# baseline (speedup 1.0000x reference)
import jax
import jax.numpy as jnp
from jax import lax
from jax.experimental import pallas as pl
from jax.experimental.pallas import tpu as pltpu
from jax.experimental.pallas import tpu_sc as plsc

D = 64
SCALE = 8.0
NC, NS = 2, 16
NW = NC * NS
BATCH = 4096
TOK = 200
VIEW_ROWS = 500_000
CHUNK = 128
NT = TOK


def _body(ids_hbm, tbl_hbm, out_hbm, ids_v, x0, x1, g0, g1, o0, o1,
          gs0, gs1, os0, os1):
    idx2 = (x0, x1)
    gbuf = (g0, g1)
    obuf = (o0, o1)
    gsem = (gs0, gs1)
    osem = (os0, os1)
    c = lax.axis_index("c")
    s = lax.axis_index("s")
    w = c * NS + s
    col0 = w * CHUNK

    pltpu.sync_copy(ids_hbm.at[:, pl.ds(col0, CHUNK)], ids_v)

    def prep_gather(j, b):
        def grp(gi, carry):
            sl = pl.ds(gi * 16, 16)
            idx2[b][sl] = lax.shift_right_logical(ids_v[j, sl], 1)
            return carry
        lax.fori_loop(0, CHUNK // 16, grp, 0)
        pltpu.async_copy(tbl_hbm.at[idx2[b]], gbuf[b], gsem[b])

    def wait_gather(j, b):
        pltpu.make_async_copy(tbl_hbm.at[idx2[b]], gbuf[b], gsem[b]).wait()

    def start_out(j, b):
        pltpu.async_copy(obuf[b], out_hbm.at[j, :, pl.ds(col0, CHUNK)],
                         osem[b])

    def wait_out(j, b):
        pltpu.make_async_copy(obuf[b], out_hbm.at[j, :, pl.ds(col0, CHUNK)],
                              osem[b]).wait()

    iota = lax.iota(jnp.int32, 16)

    def transpose_scale(j, b):
        def grp(gi, carry):
            sl = pl.ds(gi * 16, 16)
            idsv = ids_v[j, sl]
            colbase = (idsv & 1) * D
            mv = jnp.where(idsv != 0, jnp.float32(SCALE), jnp.float32(0.0))
            rows = gi * 16 + iota
            for col in range(D):
                v = plsc.load_gather(gbuf[b], [rows, colbase + col])
                obuf[b][col, sl] = v * mv
            return carry
        lax.fori_loop(0, CHUNK // 16, grp, 0)

    prep_gather(0, 0)

    def step(g, carry):
        for b in range(2):
            j = g * 2 + b

            @pl.when(j + 1 < NT)
            def _():
                prep_gather(j + 1, 1 - b)

            wait_gather(j, b)

            @pl.when(j >= 2)
            def _():
                wait_out(j - 2, b)

            transpose_scale(j, b)
            start_out(j, b)
        return carry

    lax.fori_loop(0, NT // 2, step, 0, unroll=False)
    wait_out(NT - 2, 0)
    wait_out(NT - 1, 1)


@jax.jit
def _run(ids_t, tbl_view):
    mesh = plsc.VectorSubcoreMesh(core_axis_name="c", subcore_axis_name="s")
    f = pl.kernel(
        _body,
        out_type=jax.ShapeDtypeStruct((TOK, D, BATCH), jnp.float32),
        mesh=mesh,
        compiler_params=pltpu.CompilerParams(needs_layout_passes=False,
                                             use_tc_tiling_on_sc=True),
        scratch_types=(
            [pltpu.VMEM((NT, CHUNK), jnp.int32)]
            + [pltpu.VMEM((CHUNK,), jnp.int32)] * 2
            + [pltpu.VMEM((CHUNK, 2 * D), jnp.float32)] * 2
            + [pltpu.VMEM((D, CHUNK), jnp.float32)] * 2
            + [pltpu.SemaphoreType.DMA] * 4
        ),
    )
    return f(ids_t, tbl_view)


def kernel(input, lookup_table):
    ids_t = input.astype(jnp.int32).T
    tbl_view = lookup_table.reshape(VIEW_ROWS, 2 * D)
    out_p = _run(ids_t, tbl_view)
    return out_p.transpose(2, 0, 1)


# speedup vs baseline: 1.6020x; 1.6020x over previous
import jax
import jax.numpy as jnp
from jax import lax
from jax.experimental import pallas as pl
from jax.experimental.pallas import tpu as pltpu
from jax.experimental.pallas import tpu_sc as plsc

D = 64
SCALE = 8.0
NC, NS = 2, 16
NW = NC * NS
BATCH = 4096
TOK = 200
VIEW_ROWS = 500_000
CHUNK = 128
NT = TOK


def _body(ids_hbm, tbl_hbm, out_hbm, ids_v, x0, x1, g0, g1, o0, o1,
          gs0, gs1, os0, os1):
    idx2 = (x0, x1)
    gbuf = (g0, g1)
    obuf = (o0, o1)
    gsem = (gs0, gs1)
    osem = (os0, os1)
    c = lax.axis_index("c")
    s = lax.axis_index("s")
    w = c * NS + s
    col0 = w * CHUNK

    pltpu.sync_copy(ids_hbm.at[:, pl.ds(col0, CHUNK)], ids_v)

    def prep_gather(j, b):
        def grp(gi, carry):
            sl = pl.ds(gi * 16, 16)
            idx2[b][sl] = lax.shift_right_logical(ids_v[j, sl], 1)
            return carry
        lax.fori_loop(0, CHUNK // 16, grp, 0)
        pltpu.async_copy(tbl_hbm.at[idx2[b]], gbuf[b], gsem[b])

    def wait_gather(j, b):
        pltpu.make_async_copy(tbl_hbm.at[idx2[b]], gbuf[b], gsem[b]).wait()

    def start_out(j, b):
        pltpu.async_copy(obuf[b], out_hbm.at[j, :, pl.ds(col0, CHUNK)],
                         osem[b])

    def wait_out(j, b):
        pltpu.make_async_copy(obuf[b], out_hbm.at[j, :, pl.ds(col0, CHUNK)],
                              osem[b]).wait()

    iota = lax.iota(jnp.int32, 16)

    def transpose_scale(j, b):
        def grp(gi, carry):
            sl = pl.ds(gi * 16, 16)
            idsv = ids_v[j, sl]
            colbase = (idsv & 1) * D
            mv = jnp.where(idsv != 0, jnp.float32(SCALE), jnp.float32(0.0))
            rows = gi * 16 + iota
            for col in range(D):
                chan = (col + iota) & (D - 1)
                v = plsc.load_gather(gbuf[b], [rows, colbase + chan])
                plsc.store_scatter(obuf[b], [chan, rows], v * mv)
            return carry
        lax.fori_loop(0, CHUNK // 16, grp, 0)

    prep_gather(0, 0)

    def step(g, carry):
        for b in range(2):
            j = g * 2 + b

            @pl.when(j + 1 < NT)
            def _():
                prep_gather(j + 1, 1 - b)

            wait_gather(j, b)

            @pl.when(j >= 2)
            def _():
                wait_out(j - 2, b)

            transpose_scale(j, b)
            start_out(j, b)
        return carry

    lax.fori_loop(0, NT // 2, step, 0, unroll=False)
    wait_out(NT - 2, 0)
    wait_out(NT - 1, 1)


@jax.jit
def _run(ids_t, tbl_view):
    mesh = plsc.VectorSubcoreMesh(core_axis_name="c", subcore_axis_name="s")
    f = pl.kernel(
        _body,
        out_type=jax.ShapeDtypeStruct((TOK, D, BATCH), jnp.float32),
        mesh=mesh,
        compiler_params=pltpu.CompilerParams(needs_layout_passes=False,
                                             use_tc_tiling_on_sc=True),
        scratch_types=(
            [pltpu.VMEM((NT, CHUNK), jnp.int32)]
            + [pltpu.VMEM((CHUNK,), jnp.int32)] * 2
            + [pltpu.VMEM((CHUNK, 2 * D), jnp.float32)] * 2
            + [pltpu.VMEM((D, CHUNK), jnp.float32)] * 2
            + [pltpu.SemaphoreType.DMA] * 4
        ),
    )
    return f(ids_t, tbl_view)


def kernel(input, lookup_table):
    ids_t = input.astype(jnp.int32).T
    tbl_view = lookup_table.reshape(VIEW_ROWS, 2 * D)
    out_p = _run(ids_t, tbl_view)
    return out_p.transpose(2, 0, 1)
